# Initial kernel scaffold; baseline (speedup 1.0000x reference)
import functools

import jax
import jax.numpy as jnp
from jax import lax
from jax.experimental import pallas as pl
from jax.experimental.pallas import tpu as pltpu
from jax.experimental.pallas import tpu_sc as plsc

_N = 10000
_E = 320000
_H = 128
_NTILES = 32
_EPT = _E // _NTILES
_CH = 80
_NCH = _EPT // _CH
_RPT = 632
_NPAD = _RPT * 16


def _sc_scatter(h, src3, dst3, zeros):
  mesh = plsc.VectorSubcoreMesh(core_axis_name="c", subcore_axis_name="s")

  @functools.partial(
      pl.kernel,
      out_type=jax.ShapeDtypeStruct((2 * _NPAD, _H), jnp.float32),
      mesh=mesh,
      scratch_types=[
          pltpu.VMEM((_NCH, _CH), jnp.int32),
          pltpu.VMEM((_NCH, _CH), jnp.int32),
          pltpu.VMEM((_CH, _H), jnp.float32),
          pltpu.VMEM_SHARED((_NPAD, _H), jnp.float32),
      ],
  )
  def k(h_hbm, src_hbm, dst_hbm, zeros_hbm, out_hbm, src_v, dst_v, rows, acc):
    cid = lax.axis_index("c")
    sid = lax.axis_index("s")
    wid = cid * 16 + sid
    pltpu.sync_copy(zeros_hbm, acc.at[pl.ds(sid * _RPT, _RPT)])
    pltpu.sync_copy(src_hbm.at[wid], src_v)
    pltpu.sync_copy(dst_hbm.at[wid], dst_v)
    plsc.subcore_barrier()

    @pl.loop(0, _NCH)
    def _(j):
      pltpu.sync_copy(h_hbm.at[src_v.at[j]], rows)
      pltpu.sync_copy(rows, acc.at[dst_v.at[j]], add=True)

    plsc.subcore_barrier()
    pltpu.sync_copy(acc.at[pl.ds(sid * _RPT, _RPT)],
                    out_hbm.at[pl.ds(cid * _NPAD + sid * _RPT, _RPT)])

  return k(h, src3, dst3, zeros)


def _combine_body(p0_ref, p1_ref, h_ref, wrel_ref, b_ref, wroot_ref, o_ref,
                  *, relu):
  agg = p0_ref[...] + p1_ref[...]
  acc = jnp.dot(agg, wrel_ref[...], preferred_element_type=jnp.float32)
  acc = acc + jnp.dot(h_ref[...], wroot_ref[...],
                      preferred_element_type=jnp.float32)
  acc = acc + b_ref[...]
  o_ref[...] = jnp.maximum(acc, 0.0) if relu else acc


def _tc_combine(p0, p1, h, W_rel, b, W_root, relu):
  bn = 1000
  return pl.pallas_call(
      functools.partial(_combine_body, relu=relu),
      grid=(_N // bn,),
      in_specs=[
          pl.BlockSpec((bn, _H), lambda i: (i, 0)),
          pl.BlockSpec((bn, _H), lambda i: (i, 0)),
          pl.BlockSpec((bn, _H), lambda i: (i, 0)),
          pl.BlockSpec((_H, _H), lambda i: (0, 0)),
          pl.BlockSpec((1, _H), lambda i: (0, 0)),
          pl.BlockSpec((_H, _H), lambda i: (0, 0)),
      ],
      out_specs=pl.BlockSpec((bn, _H), lambda i: (i, 0)),
      out_shape=jax.ShapeDtypeStruct((_N, _H), jnp.float32),
  )(p0, p1, h, W_rel, b.reshape(1, _H), W_root)


def kernel(x, edge_index, W_rel1, b_rel1, W_root1, W_rel2, b_rel2, W_root2,
           W_rel3, b_rel3, W_root3):
  c = W_rel3.shape[1]
  src3 = edge_index[0].reshape(_NTILES, _NCH, _CH)
  dst3 = edge_index[1].reshape(_NTILES, _NCH, _CH)
  zeros = jnp.zeros((_RPT, _H), jnp.float32)
  W_rel3p = jnp.pad(W_rel3, ((0, 0), (0, _H - c)))
  W_root3p = jnp.pad(W_root3, ((0, 0), (0, _H - c)))
  b_rel3p = jnp.pad(b_rel3, ((0, _H - c),))

  p = _sc_scatter(x, src3, dst3, zeros)
  h1 = _tc_combine(p[:_N], p[_NPAD:_NPAD + _N], x,
                   W_rel1, b_rel1, W_root1, relu=True)
  p = _sc_scatter(h1, src3, dst3, zeros)
  h2 = _tc_combine(p[:_N], p[_NPAD:_NPAD + _N], h1,
                   W_rel2, b_rel2, W_root2, relu=True)
  p = _sc_scatter(h2, src3, dst3, zeros)
  out = _tc_combine(p[:_N], p[_NPAD:_NPAD + _N], h2,
                    W_rel3p, b_rel3p, W_root3p, relu=False)
  return out[:, :c]


# speedup vs baseline: 6.9517x; 6.9517x over previous
import functools

import jax
import jax.numpy as jnp
from jax import lax
from jax.experimental import pallas as pl
from jax.experimental.pallas import tpu as pltpu
from jax.experimental.pallas import tpu_sc as plsc

_N = 10000
_E = 320000
_H = 128
_NTILES = 32
_EPT = _E // _NTILES
_CH = 80
_NCH = _EPT // _CH
_RPT = 632
_NPAD = _RPT * 16


def _sc_scatter(h, src3, dst3, zeros):
  mesh = plsc.VectorSubcoreMesh(core_axis_name="c", subcore_axis_name="s",
                                num_cores=2, num_subcores=16)

  @functools.partial(
      pl.kernel,
      out_type=jax.ShapeDtypeStruct((2 * _NPAD, _H), jnp.float32),
      mesh=mesh,
      scratch_types=[
          pltpu.VMEM((_NCH, _CH), jnp.int32),
          pltpu.VMEM((_NCH, _CH), jnp.int32),
          pltpu.VMEM((_CH, _H), jnp.float32),
          pltpu.VMEM_SHARED((_NPAD, _H), jnp.float32),
      ],
  )
  def k(h_hbm, src_hbm, dst_hbm, zeros_hbm, out_hbm, src_v, dst_v, rows, acc):
    cid = lax.axis_index("c")
    sid = lax.axis_index("s")
    wid = cid * 16 + sid
    pltpu.sync_copy(zeros_hbm, acc.at[pl.ds(sid * _RPT, _RPT)])
    pltpu.sync_copy(src_hbm.at[wid], src_v)
    pltpu.sync_copy(dst_hbm.at[wid], dst_v)
    plsc.subcore_barrier()

    @pl.loop(0, _NCH)
    def _(j):
      pltpu.sync_copy(h_hbm.at[src_v.at[j]], rows)
      pltpu.sync_copy(rows, acc.at[dst_v.at[j]], add=True)

    plsc.subcore_barrier()
    pltpu.sync_copy(acc.at[pl.ds(sid * _RPT, _RPT)],
                    out_hbm.at[pl.ds(cid * _NPAD + sid * _RPT, _RPT)])

  return k(h, src3, dst3, zeros)


def _combine_body(p0_ref, p1_ref, h_ref, wrel_ref, b_ref, wroot_ref, o_ref,
                  *, relu):
  agg = p0_ref[...] + p1_ref[...]
  acc = jnp.dot(agg, wrel_ref[...], preferred_element_type=jnp.float32)
  acc = acc + jnp.dot(h_ref[...], wroot_ref[...],
                      preferred_element_type=jnp.float32)
  acc = acc + b_ref[...]
  o_ref[...] = jnp.maximum(acc, 0.0) if relu else acc


def _tc_combine(p0, p1, h, W_rel, b, W_root, relu):
  bn = 1000
  return pl.pallas_call(
      functools.partial(_combine_body, relu=relu),
      grid=(_N // bn,),
      in_specs=[
          pl.BlockSpec((bn, _H), lambda i: (i, 0)),
          pl.BlockSpec((bn, _H), lambda i: (i, 0)),
          pl.BlockSpec((bn, _H), lambda i: (i, 0)),
          pl.BlockSpec((_H, _H), lambda i: (0, 0)),
          pl.BlockSpec((1, _H), lambda i: (0, 0)),
          pl.BlockSpec((_H, _H), lambda i: (0, 0)),
      ],
      out_specs=pl.BlockSpec((bn, _H), lambda i: (i, 0)),
      out_shape=jax.ShapeDtypeStruct((_N, _H), jnp.float32),
  )(p0, p1, h, W_rel, b.reshape(1, _H), W_root)


def kernel(x, edge_index, W_rel1, b_rel1, W_root1, W_rel2, b_rel2, W_root2,
           W_rel3, b_rel3, W_root3):
  c = W_rel3.shape[1]
  src3 = edge_index[0].reshape(_NTILES, _NCH, _CH)
  dst3 = edge_index[1].reshape(_NTILES, _NCH, _CH)
  zeros = jnp.zeros((_RPT, _H), jnp.float32)
  W_rel3p = jnp.pad(W_rel3, ((0, 0), (0, _H - c)))
  W_root3p = jnp.pad(W_root3, ((0, 0), (0, _H - c)))
  b_rel3p = jnp.pad(b_rel3, ((0, _H - c),))

  p = _sc_scatter(x, src3, dst3, zeros)
  h1 = _tc_combine(p[:_N], p[_NPAD:_NPAD + _N], x,
                   W_rel1, b_rel1, W_root1, relu=True)
  p = _sc_scatter(h1, src3, dst3, zeros)
  h2 = _tc_combine(p[:_N], p[_NPAD:_NPAD + _N], h1,
                   W_rel2, b_rel2, W_root2, relu=True)
  p = _sc_scatter(h2, src3, dst3, zeros)
  out = _tc_combine(p[:_N], p[_NPAD:_NPAD + _N], h2,
                    W_rel3p, b_rel3p, W_root3p, relu=False)
  return out[:, :c]


# speedup vs baseline: 10.3686x; 1.4915x over previous
import functools

import jax
import jax.numpy as jnp
from jax import lax
from jax.experimental import pallas as pl
from jax.experimental.pallas import tpu as pltpu
from jax.experimental.pallas import tpu_sc as plsc

_N = 10000
_E = 320000
_H = 128
_NTILES = 32
_EPT = _E // _NTILES
_CH = 80
_NCH = _EPT // _CH
_RPT = 632
_NPAD = _RPT * 16
_SB = 25
_NSB = _NCH // _SB


def _sc_scatter(h, src3, dst3, zeros):
  mesh = plsc.VectorSubcoreMesh(core_axis_name="c", subcore_axis_name="s",
                                num_cores=2, num_subcores=16)

  @functools.partial(
      pl.kernel,
      out_type=jax.ShapeDtypeStruct((2 * _NPAD, _H), jnp.float32),
      mesh=mesh,
      scratch_types=[
          pltpu.VMEM((_SB, _CH), jnp.int32),
          pltpu.VMEM((_SB, _CH), jnp.int32),
          pltpu.VMEM((_CH, _H), jnp.float32),
          pltpu.VMEM((_CH, _H), jnp.float32),
          pltpu.VMEM_SHARED((_NPAD, _H), jnp.float32),
          pltpu.SemaphoreType.DMA,
          pltpu.SemaphoreType.DMA,
      ],
  )
  def k(h_hbm, src_hbm, dst_hbm, zeros_hbm, out_hbm, src_v, dst_v,
        rows0, rows1, acc, sem0, sem1):
    cid = lax.axis_index("c")
    sid = lax.axis_index("s")
    wid = cid * 16 + sid
    pltpu.sync_copy(zeros_hbm, acc.at[pl.ds(sid * _RPT, _RPT)])
    plsc.subcore_barrier()

    def start_gather(j, buf, sem):
      pltpu.async_copy(h_hbm.at[src_v.at[j]], buf, sem)

    def finish_gather(j, buf, sem):
      pltpu.make_async_copy(h_hbm.at[src_v.at[j]], buf, sem).wait()

    @pl.loop(0, _NSB)
    def _(sb):
      pltpu.sync_copy(src_hbm.at[wid, sb], src_v)
      pltpu.sync_copy(dst_hbm.at[wid, sb], dst_v)
      start_gather(0, rows0, sem0)

      @pl.loop(0, _SB - 1, step=2)
      def _(j):
        start_gather(j + 1, rows1, sem1)
        finish_gather(j, rows0, sem0)
        pltpu.sync_copy(rows0, acc.at[dst_v.at[j]], add=True)
        start_gather(j + 2, rows0, sem0)
        finish_gather(j + 1, rows1, sem1)
        pltpu.sync_copy(rows1, acc.at[dst_v.at[j + 1]], add=True)

      finish_gather(_SB - 1, rows0, sem0)
      pltpu.sync_copy(rows0, acc.at[dst_v.at[_SB - 1]], add=True)

    plsc.subcore_barrier()
    pltpu.sync_copy(acc.at[pl.ds(sid * _RPT, _RPT)],
                    out_hbm.at[pl.ds(cid * _NPAD + sid * _RPT, _RPT)])

  return k(h, src3, dst3, zeros)


def _combine_body(p0_ref, p1_ref, h_ref, wrel_ref, b_ref, wroot_ref, o_ref,
                  *, relu):
  agg = p0_ref[...] + p1_ref[...]
  acc = jnp.dot(agg, wrel_ref[...], preferred_element_type=jnp.float32)
  acc = acc + jnp.dot(h_ref[...], wroot_ref[...],
                      preferred_element_type=jnp.float32)
  acc = acc + b_ref[...]
  o_ref[...] = jnp.maximum(acc, 0.0) if relu else acc


def _tc_combine(p0, p1, h, W_rel, b, W_root, relu):
  bn = 1000
  return pl.pallas_call(
      functools.partial(_combine_body, relu=relu),
      grid=(_N // bn,),
      in_specs=[
          pl.BlockSpec((bn, _H), lambda i: (i, 0)),
          pl.BlockSpec((bn, _H), lambda i: (i, 0)),
          pl.BlockSpec((bn, _H), lambda i: (i, 0)),
          pl.BlockSpec((_H, _H), lambda i: (0, 0)),
          pl.BlockSpec((1, _H), lambda i: (0, 0)),
          pl.BlockSpec((_H, _H), lambda i: (0, 0)),
      ],
      out_specs=pl.BlockSpec((bn, _H), lambda i: (i, 0)),
      out_shape=jax.ShapeDtypeStruct((_N, _H), jnp.float32),
  )(p0, p1, h, W_rel, b.reshape(1, _H), W_root)


def kernel(x, edge_index, W_rel1, b_rel1, W_root1, W_rel2, b_rel2, W_root2,
           W_rel3, b_rel3, W_root3):
  c = W_rel3.shape[1]
  src3 = edge_index[0].reshape(_NTILES, _NSB, _SB, _CH)
  dst3 = edge_index[1].reshape(_NTILES, _NSB, _SB, _CH)
  zeros = jnp.zeros((_RPT, _H), jnp.float32)
  W_rel3p = jnp.pad(W_rel3, ((0, 0), (0, _H - c)))
  W_root3p = jnp.pad(W_root3, ((0, 0), (0, _H - c)))
  b_rel3p = jnp.pad(b_rel3, ((0, _H - c),))

  p = _sc_scatter(x, src3, dst3, zeros)
  h1 = _tc_combine(p[:_N], p[_NPAD:_NPAD + _N], x,
                   W_rel1, b_rel1, W_root1, relu=True)
  p = _sc_scatter(h1, src3, dst3, zeros)
  h2 = _tc_combine(p[:_N], p[_NPAD:_NPAD + _N], h1,
                   W_rel2, b_rel2, W_root2, relu=True)
  p = _sc_scatter(h2, src3, dst3, zeros)
  out = _tc_combine(p[:_N], p[_NPAD:_NPAD + _N], h2,
                    W_rel3p, b_rel3p, W_root3p, relu=False)
  return out[:, :c]


# speedup vs baseline: 13.0064x; 1.2544x over previous
import functools

import jax
import jax.numpy as jnp
from jax import lax
from jax.experimental import pallas as pl
from jax.experimental.pallas import tpu as pltpu
from jax.experimental.pallas import tpu_sc as plsc

_N = 10000
_E = 320000
_H = 128
_NTILES = 32
_EPT = _E // _NTILES
_CH = 80
_NCH = _EPT // _CH
_RPT = 632
_NPAD = _RPT * 16
_SB = 25
_NSB = _NCH // _SB


def _sc_scatter(h, ei, zeros):
  mesh = plsc.VectorSubcoreMesh(core_axis_name="c", subcore_axis_name="s",
                                num_cores=2, num_subcores=16)
  part = jax.ShapeDtypeStruct((_NPAD, _H), jnp.float32)

  @functools.partial(
      pl.kernel,
      out_type=[part, part],
      mesh=mesh,
      scratch_types=[
          pltpu.VMEM((_SB, _CH), jnp.int32),
          pltpu.VMEM((_SB, _CH), jnp.int32),
          pltpu.VMEM((_CH, _H), jnp.float32),
          pltpu.VMEM((_CH, _H), jnp.float32),
          pltpu.VMEM((_CH, _H), jnp.float32),
          pltpu.VMEM_SHARED((_NPAD, _H), jnp.float32),
          pltpu.SemaphoreType.DMA,
          pltpu.SemaphoreType.DMA,
          pltpu.SemaphoreType.DMA,
      ],
  )
  def k(h_hbm, ei_hbm, zeros_hbm, out0_hbm, out1_hbm, src_v, dst_v,
        rows0, rows1, rows2, acc, sem0, sem1, sem2):
    cid = lax.axis_index("c")
    sid = lax.axis_index("s")
    wid = cid * 16 + sid
    bufs = ((rows0, sem0), (rows1, sem1), (rows2, sem2))
    pltpu.sync_copy(zeros_hbm, acc.at[pl.ds(sid * _RPT, _RPT)])
    plsc.subcore_barrier()

    def start_gather(j, buf, sem):
      pltpu.async_copy(h_hbm.at[src_v.at[j]], buf, sem)

    def finish_gather(j, buf, sem):
      pltpu.make_async_copy(h_hbm.at[src_v.at[j]], buf, sem).wait()

    @pl.loop(0, _NSB)
    def _(sb):
      pltpu.sync_copy(ei_hbm.at[0, wid, sb], src_v)
      pltpu.sync_copy(ei_hbm.at[1, wid, sb], dst_v)
      for b in range(3):
        start_gather(b, *bufs[b])

      @pl.loop(0, _SB - 1, step=3)
      def _(j):
        for b in range(3):
          finish_gather(j + b, *bufs[b])
          pltpu.sync_copy(bufs[b][0], acc.at[dst_v.at[j + b]], add=True)

          @pl.when(j + b + 3 < _SB)
          def _():
            start_gather(j + b + 3, *bufs[b])

      finish_gather(_SB - 1, *bufs[0])
      pltpu.sync_copy(bufs[0][0], acc.at[dst_v.at[_SB - 1]], add=True)

    plsc.subcore_barrier()
    my_rows = acc.at[pl.ds(sid * _RPT, _RPT)]

    @pl.when(cid == 0)
    def _():
      pltpu.sync_copy(my_rows, out0_hbm.at[pl.ds(sid * _RPT, _RPT)])

    @pl.when(cid == 1)
    def _():
      pltpu.sync_copy(my_rows, out1_hbm.at[pl.ds(sid * _RPT, _RPT)])

  return k(h, ei, zeros)


def _combine_body(p0_ref, p1_ref, h_ref, wrel_ref, b_ref, wroot_ref, o_ref,
                  *, relu):
  agg = p0_ref[...] + p1_ref[...]
  acc = jnp.dot(agg, wrel_ref[...], preferred_element_type=jnp.float32)
  acc = acc + jnp.dot(h_ref[...], wroot_ref[...],
                      preferred_element_type=jnp.float32)
  acc = acc + b_ref[...]
  o_ref[...] = jnp.maximum(acc, 0.0) if relu else acc


def _tc_combine(p0, p1, h, W_rel, b, W_root, relu, out_w=None):
  bn = 2000
  ow = _H if out_w is None else out_w
  return pl.pallas_call(
      functools.partial(_combine_body, relu=relu),
      grid=(_N // bn,),
      in_specs=[
          pl.BlockSpec((bn, _H), lambda i: (i, 0)),
          pl.BlockSpec((bn, _H), lambda i: (i, 0)),
          pl.BlockSpec((bn, _H), lambda i: (i, 0)),
          pl.BlockSpec((_H, ow), lambda i: (0, 0)),
          pl.BlockSpec((1, ow), lambda i: (0, 0)),
          pl.BlockSpec((_H, ow), lambda i: (0, 0)),
      ],
      out_specs=pl.BlockSpec((bn, ow), lambda i: (i, 0)),
      out_shape=jax.ShapeDtypeStruct((_N, ow), jnp.float32),
  )(p0, p1, h, W_rel, b.reshape(1, ow), W_root)


def kernel(x, edge_index, W_rel1, b_rel1, W_root1, W_rel2, b_rel2, W_root2,
           W_rel3, b_rel3, W_root3):
  c = W_rel3.shape[1]
  ei = edge_index.reshape(2, _NTILES, _NSB, _SB, _CH)
  zeros = jnp.zeros((_RPT, _H), jnp.float32)

  p0, p1 = _sc_scatter(x, ei, zeros)
  h1 = _tc_combine(p0, p1, x, W_rel1, b_rel1, W_root1, relu=True)
  p0, p1 = _sc_scatter(h1, ei, zeros)
  h2 = _tc_combine(p0, p1, h1, W_rel2, b_rel2, W_root2, relu=True)
  p0, p1 = _sc_scatter(h2, ei, zeros)
  out = _tc_combine(p0, p1, h2, W_rel3, b_rel3, W_root3,
                    relu=False, out_w=c)
  return out
